# baseline (speedup 1.0000x reference)
import jax
import jax.numpy as jnp
from jax import lax
from jax.experimental import pallas as pl
from jax.experimental.pallas import tpu as pltpu
from jax.experimental.pallas import tpu_sc as plsc

C_IN = 100000
D_MODEL = 64
BATCH = 4096
HIST = 50

NC = 2
NS = 16
NW = NC * NS

R_PER_W = BATCH // NW
NBUF = 8
LA = 4
INNER = NBUF


def _gather_body(x_hbm, table_hbm, out_hbm, idx_v, rows, gsems, osems):
    wid = lax.axis_index("s") * NC + lax.axis_index("c")
    base = wid * R_PER_W
    pltpu.sync_copy(x_hbm.at[pl.ds(base, R_PER_W)], idx_v)

    def gather(r, b):
        return pltpu.make_async_copy(
            table_hbm.at[idx_v.at[r]], rows[b], gsems[b]
        )

    def writeback(r, b):
        return pltpu.make_async_copy(rows[b], out_hbm.at[base + r], osems[b])

    for r in range(LA):
        gather(r, r).start()

    @pl.loop(0, R_PER_W // INNER)
    def _outer(p):
        r0 = p * INNER
        for t in range(INNER):
            r = r0 + t
            nb = (t + LA) % NBUF
            if t >= NBUF - LA:
                writeback(r + LA - NBUF, nb).wait()
            else:
                @pl.when(p > 0)
                def _():
                    writeback(r + LA - NBUF, nb).wait()

            @pl.when(r + LA < R_PER_W)
            def _():
                gather(r + LA, nb).start()

            gather(r, t).wait()
            writeback(r, t).start()

    for t in range(NBUF - LA):
        rt = R_PER_W - (NBUF - LA) + t
        writeback(rt, rt % NBUF).wait()


@jax.jit
def kernel(x, W):
    mesh = plsc.VectorSubcoreMesh(core_axis_name="c", subcore_axis_name="s")
    return pl.kernel(
        _gather_body,
        out_type=jax.ShapeDtypeStruct((BATCH, HIST, D_MODEL), jnp.float32),
        mesh=mesh,
        scratch_types=[
            pltpu.VMEM((R_PER_W, HIST), jnp.int32),
            tuple(pltpu.VMEM((HIST, D_MODEL), jnp.float32) for _ in range(NBUF)),
            tuple(pltpu.SemaphoreType.DMA for _ in range(NBUF)),
            tuple(pltpu.SemaphoreType.DMA for _ in range(NBUF)),
        ],
        compiler_params=pltpu.CompilerParams(use_tc_tiling_on_sc=False),
    )(x, W)
